# baseline (speedup 1.0000x reference)
import jax
import jax.numpy as jnp
from jax.experimental import pallas as pl
from jax.experimental.pallas import tpu as pltpu

N_E = 1024
E_DIM = 256
BETA = 0.25


def _vq_body(z_ref, emb_ref, rn_ref, zq_ref, idx_ref, loss_ref):
    emb = emb_ref[...]

    cn = jnp.sum(emb * emb, axis=1, keepdims=True)

    nimg = z_ref.shape[0]
    parts = []
    for i in range(nimg):
        z = z_ref[i]
        rn = rn_ref[i]

        s = jax.lax.dot_general(emb, z, (((1,), (0,)), ((), ())),
                                preferred_element_type=jnp.float32)

        d = (rn + cn) - 2.0 * s

        iota = jax.lax.broadcasted_iota(jnp.int32, (N_E, z.shape[1]), 0)
        m = jnp.min(d, axis=0, keepdims=True)
        idx = jnp.min(jnp.where(d == m, iota, N_E),
                      axis=0).astype(jnp.int32)

        onehot = (iota == idx[None, :]).astype(jnp.float32)
        zq = jax.lax.dot_general(emb, onehot, (((0,), (0,)), ((), ())),
                                 preferred_element_type=jnp.float32)

        zq_ref[i] = zq
        idx_ref[i, 0] = idx
        parts.append(jnp.sum(m, axis=1, keepdims=True))

    loss_ref[0] = sum(parts)


def kernel(z, emb):
    B, C, H, W = z.shape
    HW = H * W
    zr = z.reshape(B, C, HW)

    rn = jnp.sum(z * z, axis=1).reshape(B, 1, HW)

    NI = 4
    NB = B // NI
    zq, idx, loss_p = pl.pallas_call(
        _vq_body,
        grid=(NB,),
        in_specs=[
            pl.BlockSpec((NI, C, HW), lambda b: (b, 0, 0)),
            pl.BlockSpec((N_E, E_DIM), lambda b: (0, 0)),
            pl.BlockSpec((NI, 1, HW), lambda b: (b, 0, 0)),
        ],
        out_specs=[
            pl.BlockSpec((NI, C, HW), lambda b: (b, 0, 0)),
            pl.BlockSpec((NI, 1, HW), lambda b: (b, 0, 0)),
            pl.BlockSpec((1, 1, 1), lambda b: (b, 0, 0)),
        ],
        out_shape=[
            jax.ShapeDtypeStruct((B, C, HW), jnp.float32),
            jax.ShapeDtypeStruct((B, 1, HW), jnp.int32),
            jax.ShapeDtypeStruct((NB, 1, 1), jnp.float32),
        ],
        compiler_params=pltpu.CompilerParams(
            dimension_semantics=("parallel",)),
    )(zr, emb, rn)

    z_q_out = zq.reshape(B, C, H, W)
    min_encoding_indices = idx.reshape(-1)
    loss = BETA * (jnp.sum(loss_p) / (B * C * HW))
    return (z_q_out, loss, (None, None, min_encoding_indices))


if __name__ == "__main__":
    import numpy as np
    import reference as _r

    d = _r.setup_inputs(0)
    with pltpu.force_tpu_interpret_mode():
        out = kernel(d["z"], d["emb"])
    ref = _r.reference(d["z"], d["emb"])
    for a, b in zip(jax.tree.leaves(out), jax.tree.leaves(ref)):
        a, b = np.asarray(a, np.float64), np.asarray(b, np.float64)
        print(a.shape, a.dtype, "rvr=",
              float(((a - b) ** 2).mean() / max(float((b ** 2).mean()), 1e-12)))


# speedup vs baseline: 1.0073x; 1.0073x over previous
import jax
import jax.numpy as jnp
from jax.experimental import pallas as pl
from jax.experimental.pallas import tpu as pltpu

N_E = 1024
E_DIM = 256
BETA = 0.25


def _vq_body(z_ref, emb_ref, rn_ref, zq_ref, idx_ref, loss_ref):
    emb = emb_ref[...]

    cn = jnp.sum(emb * emb, axis=1, keepdims=True)

    nimg = z_ref.shape[0]
    parts = []
    for i in range(nimg):
        z = z_ref[i]
        rn = rn_ref[i]

        s = jax.lax.dot_general(emb, z, (((1,), (0,)), ((), ())),
                                preferred_element_type=jnp.float32)

        d = (rn + cn) - 2.0 * s

        iota = jax.lax.broadcasted_iota(jnp.int32, (N_E, z.shape[1]), 0)
        m = jnp.min(d, axis=0, keepdims=True)
        idx = jnp.min(jnp.where(d == m, iota, N_E),
                      axis=0).astype(jnp.int32)

        onehot = (iota == idx[None, :]).astype(jnp.float32)
        zq = jax.lax.dot_general(emb, onehot, (((0,), (0,)), ((), ())),
                                 preferred_element_type=jnp.float32)

        zq_ref[i] = zq
        idx_ref[i, 0] = idx
        parts.append(jnp.sum(m, axis=1, keepdims=True))

    loss_ref[0] = sum(parts)


def kernel(z, emb):
    B, C, H, W = z.shape
    HW = H * W
    zr = z.reshape(B, C, HW)

    rn = jnp.sum(z * z, axis=1).reshape(B, 1, HW)

    NI = 2
    NB = B // NI
    zq, idx, loss_p = pl.pallas_call(
        _vq_body,
        grid=(NB,),
        in_specs=[
            pl.BlockSpec((NI, C, HW), lambda b: (b, 0, 0)),
            pl.BlockSpec((N_E, E_DIM), lambda b: (0, 0)),
            pl.BlockSpec((NI, 1, HW), lambda b: (b, 0, 0)),
        ],
        out_specs=[
            pl.BlockSpec((NI, C, HW), lambda b: (b, 0, 0)),
            pl.BlockSpec((NI, 1, HW), lambda b: (b, 0, 0)),
            pl.BlockSpec((1, 1, 1), lambda b: (b, 0, 0)),
        ],
        out_shape=[
            jax.ShapeDtypeStruct((B, C, HW), jnp.float32),
            jax.ShapeDtypeStruct((B, 1, HW), jnp.int32),
            jax.ShapeDtypeStruct((NB, 1, 1), jnp.float32),
        ],
        compiler_params=pltpu.CompilerParams(
            dimension_semantics=("parallel",)),
    )(zr, emb, rn)

    z_q_out = zq.reshape(B, C, H, W)
    min_encoding_indices = idx.reshape(-1)
    loss = BETA * (jnp.sum(loss_p) / (B * C * HW))
    return (z_q_out, loss, (None, None, min_encoding_indices))
